# Initial kernel scaffold; baseline (speedup 1.0000x reference)
import jax
import jax.numpy as jnp
from jax import lax
from jax.experimental import pallas as pl
from jax.experimental.pallas import tpu as pltpu
from jax.experimental.pallas import tpu_sc as plsc

N = 10000
D = 128
E = 320000
NC = 2
NS = 16
CH = 80
NCH = E // CH
TCH = NCH // (NC * NS)
RPT = N // NS
DEGW = 16


def _sc_mesh():
    return plsc.VectorSubcoreMesh(
        core_axis_name="c", subcore_axis_name="s", num_cores=NC, num_subcores=NS
    )


def _agg_partials(h, src2d, dst2d, zeros):
    def body(h_hbm, src_hbm, dst_hbm, zero_hbm, out_hbm, src_v, dst_v, rows_v, acc, sem):
        cid = lax.axis_index("c")
        sid = lax.axis_index("s")
        tid = cid * NS + sid
        r0 = sid * RPT
        pltpu.sync_copy(zero_hbm.at[pl.ds(r0, RPT), :], acc.at[pl.ds(r0, RPT), :])
        base = tid * TCH
        pltpu.sync_copy(src_hbm.at[pl.ds(base, TCH), :], src_v)
        pltpu.sync_copy(dst_hbm.at[pl.ds(base, TCH), :], dst_v)
        plsc.subcore_barrier()

        def step(j, carry):
            pltpu.async_copy(h_hbm.at[src_v.at[j]], rows_v, sem).wait()
            pltpu.sync_copy(rows_v, acc.at[dst_v.at[j]], add=True)
            return carry

        lax.fori_loop(0, TCH, step, 0)
        plsc.subcore_barrier()
        pltpu.sync_copy(acc.at[pl.ds(r0, RPT), :], out_hbm.at[cid, pl.ds(r0, RPT), :])

    return pl.kernel(
        body,
        out_type=jax.ShapeDtypeStruct((NC, N, D), jnp.float32),
        mesh=_sc_mesh(),
        scratch_types=[
            pltpu.VMEM((TCH, CH), jnp.int32),
            pltpu.VMEM((TCH, CH), jnp.int32),
            pltpu.VMEM((CH, D), jnp.float32),
            pltpu.VMEM_SHARED((N, D), jnp.float32),
            pltpu.SemaphoreType.DMA,
        ],
    )(h, src2d, dst2d, zeros)


def _deg_partials(dst2d, ones, zeros16):
    def body(dst_hbm, ones_hbm, zero_hbm, out_hbm, dst_v, ones_v, acc, sem):
        cid = lax.axis_index("c")
        sid = lax.axis_index("s")
        tid = cid * NS + sid
        r0 = sid * RPT
        pltpu.sync_copy(zero_hbm.at[pl.ds(r0, RPT), :], acc.at[pl.ds(r0, RPT), :])
        pltpu.sync_copy(ones_hbm, ones_v)
        base = tid * TCH
        pltpu.sync_copy(dst_hbm.at[pl.ds(base, TCH), :], dst_v)
        plsc.subcore_barrier()

        def step(j, carry):
            pltpu.sync_copy(ones_v, acc.at[dst_v.at[j]], add=True)
            return carry

        lax.fori_loop(0, TCH, step, 0)
        plsc.subcore_barrier()
        pltpu.sync_copy(acc.at[pl.ds(r0, RPT), :], out_hbm.at[cid, pl.ds(r0, RPT), :])

    return pl.kernel(
        body,
        out_type=jax.ShapeDtypeStruct((NC, N, DEGW), jnp.float32),
        mesh=_sc_mesh(),
        scratch_types=[
            pltpu.VMEM((TCH, CH), jnp.int32),
            pltpu.VMEM((CH, DEGW), jnp.float32),
            pltpu.VMEM_SHARED((N, DEGW), jnp.float32),
            pltpu.SemaphoreType.DMA,
        ],
    )(dst2d, ones, zeros16)


def _dense_layer(h, accp, degp, Ws, Wn, b2d, relu):
    BLK = 400
    G = N // BLK

    def body(h_ref, a0, a1, d0, d1, ws, wn, b, out):
        deg = d0[0][:, 0:1] + d1[0][:, 0:1]
        inv = 1.0 / jnp.maximum(deg, 1.0)
        mean = (a0[0] + a1[0]) * inv
        y = jnp.dot(h_ref[...], ws[...], preferred_element_type=jnp.float32)
        y = y + jnp.dot(mean, wn[...], preferred_element_type=jnp.float32)
        y = y + b[...]
        if relu:
            y = jnp.maximum(y, 0.0)
        out[...] = y

    return pl.pallas_call(
        body,
        grid=(G,),
        in_specs=[
            pl.BlockSpec((BLK, D), lambda i: (i, 0)),
            pl.BlockSpec((1, BLK, D), lambda i: (0, i, 0)),
            pl.BlockSpec((1, BLK, D), lambda i: (1, i, 0)),
            pl.BlockSpec((1, BLK, DEGW), lambda i: (0, i, 0)),
            pl.BlockSpec((1, BLK, DEGW), lambda i: (1, i, 0)),
            pl.BlockSpec((D, D), lambda i: (0, 0)),
            pl.BlockSpec((D, D), lambda i: (0, 0)),
            pl.BlockSpec((1, D), lambda i: (0, 0)),
        ],
        out_specs=pl.BlockSpec((BLK, D), lambda i: (i, 0)),
        out_shape=jax.ShapeDtypeStruct((N, D), jnp.float32),
    )(h, accp, accp, degp, degp, Ws, Wn, b2d)


def kernel(x, edge_index, W_self_0, W_neigh_0, b_0, W_self_1, W_neigh_1, b_1,
           W_self_2, W_neigh_2, b_2):
    src2d = edge_index[0].reshape(NCH, CH)
    dst2d = edge_index[1].reshape(NCH, CH)
    zeros = jnp.zeros((N, D), jnp.float32)
    zeros16 = jnp.zeros((N, DEGW), jnp.float32)
    ones = jnp.ones((CH, DEGW), jnp.float32)
    degp = _deg_partials(dst2d, ones, zeros16)
    params = [
        (W_self_0, W_neigh_0, b_0),
        (W_self_1, W_neigh_1, b_1),
        (W_self_2, W_neigh_2, b_2),
    ]
    h = x
    for l, (Ws, Wn, b) in enumerate(params):
        accp = _agg_partials(h, src2d, dst2d, zeros)
        h = _dense_layer(h, accp, degp, Ws, Wn, b.reshape(1, D), relu=(l < 2))
    return h


# speedup vs baseline: 6.1404x; 6.1404x over previous
import jax
import jax.numpy as jnp
from jax import lax
from jax.experimental import pallas as pl
from jax.experimental.pallas import tpu as pltpu
from jax.experimental.pallas import tpu_sc as plsc

N = 10000
D = 128
E = 320000
NC = 2
NS = 16
CH = 80
NCH = E // CH
NT = NC * NS
TCH = NCH // NT
RB = 400
NB = N // RB
DEGW = 128


def _sc_mesh():
    return plsc.VectorSubcoreMesh(
        core_axis_name="c", subcore_axis_name="s", num_cores=NC, num_subcores=NS
    )


def _agg_partials(h, src3d, dst3d, zeros):
    def body(h_hbm, src_hbm, dst_hbm, zero_hbm, out_hbm, src_v, dst_v, rows_v, acc, sem):
        cid = lax.axis_index("c")
        sid = lax.axis_index("s")
        tid = cid * NS + sid
        for blk in range(NB):
            @pl.when(blk % NS == sid)
            def _():
                pltpu.sync_copy(zero_hbm.at[pl.ds(blk * RB, RB), :],
                                acc.at[pl.ds(blk * RB, RB), :])
        pltpu.sync_copy(src_hbm.at[tid], src_v)
        pltpu.sync_copy(dst_hbm.at[tid], dst_v)
        plsc.subcore_barrier()

        def step(j, carry):
            pltpu.async_copy(h_hbm.at[src_v.at[j]], rows_v, sem).wait()
            pltpu.sync_copy(rows_v, acc.at[dst_v.at[j]], add=True)
            return carry

        lax.fori_loop(0, TCH, step, 0)
        plsc.subcore_barrier()
        for blk in range(NB):
            @pl.when(blk % NS == sid)
            def _():
                pltpu.sync_copy(acc.at[pl.ds(blk * RB, RB), :],
                                out_hbm.at[cid, pl.ds(blk * RB, RB), :])

    return pl.kernel(
        body,
        out_type=jax.ShapeDtypeStruct((NC, N, D), jnp.float32),
        mesh=_sc_mesh(),
        scratch_types=[
            pltpu.VMEM((TCH, CH), jnp.int32),
            pltpu.VMEM((TCH, CH), jnp.int32),
            pltpu.VMEM((CH, D), jnp.float32),
            pltpu.VMEM_SHARED((N, D), jnp.float32),
            pltpu.SemaphoreType.DMA,
        ],
    )(h, src3d, dst3d, zeros)


def _deg_partials(dst3d, ones, zeros16):
    def body(dst_hbm, ones_hbm, zero_hbm, out_hbm, dst_v, ones_v, acc, sem):
        cid = lax.axis_index("c")
        sid = lax.axis_index("s")
        tid = cid * NS + sid
        for blk in range(NB):
            @pl.when(blk % NS == sid)
            def _():
                pltpu.sync_copy(zero_hbm.at[pl.ds(blk * RB, RB), :],
                                acc.at[pl.ds(blk * RB, RB), :])
        pltpu.sync_copy(ones_hbm, ones_v)
        pltpu.sync_copy(dst_hbm.at[tid], dst_v)
        plsc.subcore_barrier()

        def step(j, carry):
            pltpu.sync_copy(ones_v, acc.at[dst_v.at[j]], add=True)
            return carry

        lax.fori_loop(0, TCH, step, 0)
        plsc.subcore_barrier()
        for blk in range(NB):
            @pl.when(blk % NS == sid)
            def _():
                pltpu.sync_copy(acc.at[pl.ds(blk * RB, RB), :],
                                out_hbm.at[cid, pl.ds(blk * RB, RB), :])

    return pl.kernel(
        body,
        out_type=jax.ShapeDtypeStruct((NC, N, DEGW), jnp.float32),
        mesh=_sc_mesh(),
        scratch_types=[
            pltpu.VMEM((TCH, CH), jnp.int32),
            pltpu.VMEM((CH, DEGW), jnp.float32),
            pltpu.VMEM_SHARED((N, DEGW), jnp.float32),
            pltpu.SemaphoreType.DMA,
        ],
    )(dst3d, ones, zeros16)


def _dense_layer(h, accp, degp, Ws, Wn, b2d, relu):
    BLK = 400
    G = N // BLK

    def body(h_ref, a0, a1, d0, d1, ws, wn, b, out):
        deg = d0[0][:, 0:1] + d1[0][:, 0:1]
        inv = 1.0 / jnp.maximum(deg, 1.0)
        mean = (a0[0] + a1[0]) * inv
        y = jnp.dot(h_ref[...], ws[...], preferred_element_type=jnp.float32)
        y = y + jnp.dot(mean, wn[...], preferred_element_type=jnp.float32)
        y = y + b[...]
        if relu:
            y = jnp.maximum(y, 0.0)
        out[...] = y

    return pl.pallas_call(
        body,
        grid=(G,),
        in_specs=[
            pl.BlockSpec((BLK, D), lambda i: (i, 0)),
            pl.BlockSpec((1, BLK, D), lambda i: (0, i, 0)),
            pl.BlockSpec((1, BLK, D), lambda i: (1, i, 0)),
            pl.BlockSpec((1, BLK, DEGW), lambda i: (0, i, 0)),
            pl.BlockSpec((1, BLK, DEGW), lambda i: (1, i, 0)),
            pl.BlockSpec((D, D), lambda i: (0, 0)),
            pl.BlockSpec((D, D), lambda i: (0, 0)),
            pl.BlockSpec((1, D), lambda i: (0, 0)),
        ],
        out_specs=pl.BlockSpec((BLK, D), lambda i: (i, 0)),
        out_shape=jax.ShapeDtypeStruct((N, D), jnp.float32),
    )(h, accp, accp, degp, degp, Ws, Wn, b2d)


def kernel(x, edge_index, W_self_0, W_neigh_0, b_0, W_self_1, W_neigh_1, b_1,
           W_self_2, W_neigh_2, b_2):
    src3d = edge_index[0].reshape(NT, TCH, CH)
    dst3d = edge_index[1].reshape(NT, TCH, CH)
    zeros = jnp.zeros((N, D), jnp.float32)
    zeros16 = jnp.zeros((N, DEGW), jnp.float32)
    ones = jnp.ones((CH, DEGW), jnp.float32)
    degp = _deg_partials(dst3d, ones, zeros16)
    params = [
        (W_self_0, W_neigh_0, b_0),
        (W_self_1, W_neigh_1, b_1),
        (W_self_2, W_neigh_2, b_2),
    ]
    h = x
    for l, (Ws, Wn, b) in enumerate(params):
        accp = _agg_partials(h, src3d, dst3d, zeros)
        h = _dense_layer(h, accp, degp, Ws, Wn, b.reshape(1, D), relu=(l < 2))
    return h


# speedup vs baseline: 9.0363x; 1.4716x over previous
import jax
import jax.numpy as jnp
from jax import lax
from jax.experimental import pallas as pl
from jax.experimental.pallas import tpu as pltpu
from jax.experimental.pallas import tpu_sc as plsc

N = 10000
D = 128
E = 320000
NC = 2
NS = 16
CH = 80
NCH = E // CH
NT = NC * NS
TCH = NCH // NT
IDXR = 64
RB = 400
NB = N // RB
DEGW = 128


def _sc_mesh():
    return plsc.VectorSubcoreMesh(
        core_axis_name="c", subcore_axis_name="s", num_cores=NC, num_subcores=NS
    )


def _agg_partials(h, src3d, dst3d, zeros):
    def body(h_hbm, src_hbm, dst_hbm, zero_hbm, out_hbm, src_v, dst_v, rows_v, acc,
             sem_a, sem_b):
        cid = lax.axis_index("c")
        sid = lax.axis_index("s")
        tid = cid * NS + sid
        for blk in range(NB):
            @pl.when(blk % NS == sid)
            def _():
                pltpu.sync_copy(zero_hbm.at[pl.ds(blk * RB, RB), :],
                                acc.at[pl.ds(blk * RB, RB), :])
        plsc.subcore_barrier()

        rows_a = rows_v.at[0]
        rows_b = rows_v.at[1]

        def wait_gather(buf, s):
            pltpu.make_async_copy(h_hbm.at[pl.ds(0, CH), :], buf, s).wait()

        def run_round(off, cnt):
            pltpu.sync_copy(src_hbm.at[tid, pl.ds(off, cnt), :],
                            src_v.at[pl.ds(0, cnt), :])
            pltpu.sync_copy(dst_hbm.at[tid, pl.ds(off, cnt), :],
                            dst_v.at[pl.ds(0, cnt), :])
            pltpu.async_copy(h_hbm.at[src_v.at[0]], rows_a, sem_a)

            def step(jj, carry):
                j0 = 2 * jj
                pltpu.async_copy(h_hbm.at[src_v.at[j0 + 1]], rows_b, sem_b)
                wait_gather(rows_a, sem_a)
                pltpu.sync_copy(rows_a, acc.at[dst_v.at[j0]], add=True)

                @pl.when(j0 + 2 < cnt)
                def _():
                    pltpu.async_copy(h_hbm.at[src_v.at[j0 + 2]], rows_a, sem_a)

                wait_gather(rows_b, sem_b)
                pltpu.sync_copy(rows_b, acc.at[dst_v.at[j0 + 1]], add=True)
                return carry

            lax.fori_loop(0, cnt // 2, step, 0)
            if cnt % 2 == 1:
                wait_gather(rows_a, sem_a)
                pltpu.sync_copy(rows_a, acc.at[dst_v.at[cnt - 1]], add=True)

        off = 0
        while off < TCH:
            cnt = min(IDXR, TCH - off)
            run_round(off, cnt)
            off += cnt
        plsc.subcore_barrier()
        for blk in range(NB):
            @pl.when(blk % NS == sid)
            def _():
                pltpu.sync_copy(acc.at[pl.ds(blk * RB, RB), :],
                                out_hbm.at[cid, pl.ds(blk * RB, RB), :])

    return pl.kernel(
        body,
        out_type=jax.ShapeDtypeStruct((NC, N, D), jnp.float32),
        mesh=_sc_mesh(),
        scratch_types=[
            pltpu.VMEM((IDXR, CH), jnp.int32),
            pltpu.VMEM((IDXR, CH), jnp.int32),
            pltpu.VMEM((2, CH, D), jnp.float32),
            pltpu.VMEM_SHARED((N, D), jnp.float32),
            pltpu.SemaphoreType.DMA,
            pltpu.SemaphoreType.DMA,
        ],
    )(h, src3d, dst3d, zeros)


def _deg_partials(dst3d, ones, zeros16):
    def body(dst_hbm, ones_hbm, zero_hbm, out_hbm, dst_v, ones_v, acc, sem):
        cid = lax.axis_index("c")
        sid = lax.axis_index("s")
        tid = cid * NS + sid
        for blk in range(NB):
            @pl.when(blk % NS == sid)
            def _():
                pltpu.sync_copy(zero_hbm.at[pl.ds(blk * RB, RB), :],
                                acc.at[pl.ds(blk * RB, RB), :])
        pltpu.sync_copy(ones_hbm, ones_v)
        pltpu.sync_copy(dst_hbm.at[tid], dst_v)
        plsc.subcore_barrier()

        def step(j, carry):
            pltpu.sync_copy(ones_v, acc.at[dst_v.at[j]], add=True)
            return carry

        lax.fori_loop(0, TCH, step, 0)
        plsc.subcore_barrier()
        for blk in range(NB):
            @pl.when(blk % NS == sid)
            def _():
                pltpu.sync_copy(acc.at[pl.ds(blk * RB, RB), :],
                                out_hbm.at[cid, pl.ds(blk * RB, RB), :])

    return pl.kernel(
        body,
        out_type=jax.ShapeDtypeStruct((NC, N, DEGW), jnp.float32),
        mesh=_sc_mesh(),
        scratch_types=[
            pltpu.VMEM((TCH, CH), jnp.int32),
            pltpu.VMEM((CH, DEGW), jnp.float32),
            pltpu.VMEM_SHARED((N, DEGW), jnp.float32),
            pltpu.SemaphoreType.DMA,
        ],
    )(dst3d, ones, zeros16)


def _dense_layer(h, accp, degp, Ws, Wn, b2d, relu):
    BLK = 400
    G = N // BLK

    def body(h_ref, a0, a1, d0, d1, ws, wn, b, out):
        deg = d0[0][:, 0:1] + d1[0][:, 0:1]
        inv = 1.0 / jnp.maximum(deg, 1.0)
        mean = (a0[0] + a1[0]) * inv
        y = jnp.dot(h_ref[...], ws[...], preferred_element_type=jnp.float32)
        y = y + jnp.dot(mean, wn[...], preferred_element_type=jnp.float32)
        y = y + b[...]
        if relu:
            y = jnp.maximum(y, 0.0)
        out[...] = y

    return pl.pallas_call(
        body,
        grid=(G,),
        in_specs=[
            pl.BlockSpec((BLK, D), lambda i: (i, 0)),
            pl.BlockSpec((1, BLK, D), lambda i: (0, i, 0)),
            pl.BlockSpec((1, BLK, D), lambda i: (1, i, 0)),
            pl.BlockSpec((1, BLK, DEGW), lambda i: (0, i, 0)),
            pl.BlockSpec((1, BLK, DEGW), lambda i: (1, i, 0)),
            pl.BlockSpec((D, D), lambda i: (0, 0)),
            pl.BlockSpec((D, D), lambda i: (0, 0)),
            pl.BlockSpec((1, D), lambda i: (0, 0)),
        ],
        out_specs=pl.BlockSpec((BLK, D), lambda i: (i, 0)),
        out_shape=jax.ShapeDtypeStruct((N, D), jnp.float32),
    )(h, accp, accp, degp, degp, Ws, Wn, b2d)


def kernel(x, edge_index, W_self_0, W_neigh_0, b_0, W_self_1, W_neigh_1, b_1,
           W_self_2, W_neigh_2, b_2):
    src3d = edge_index[0].reshape(NT, TCH, CH)
    dst3d = edge_index[1].reshape(NT, TCH, CH)
    zeros = jnp.zeros((N, D), jnp.float32)
    zeros16 = jnp.zeros((N, DEGW), jnp.float32)
    ones = jnp.ones((CH, DEGW), jnp.float32)
    degp = _deg_partials(dst3d, ones, zeros16)
    params = [
        (W_self_0, W_neigh_0, b_0),
        (W_self_1, W_neigh_1, b_1),
        (W_self_2, W_neigh_2, b_2),
    ]
    h = x
    for l, (Ws, Wn, b) in enumerate(params):
        accp = _agg_partials(h, src3d, dst3d, zeros)
        h = _dense_layer(h, accp, degp, Ws, Wn, b.reshape(1, D), relu=(l < 2))
    return h


# speedup vs baseline: 9.0873x; 1.0056x over previous
import jax
import jax.numpy as jnp
from jax import lax
from jax.experimental import pallas as pl
from jax.experimental.pallas import tpu as pltpu
from jax.experimental.pallas import tpu_sc as plsc

N = 10000
D = 128
E = 320000
NC = 2
NS = 16
CH = 80
NCH = E // CH
NT = NC * NS
TCH = NCH // NT
IDXR = 32
NBUF = 4
RB = 400
NB = N // RB
DEGW = 128


def _sc_mesh():
    return plsc.VectorSubcoreMesh(
        core_axis_name="c", subcore_axis_name="s", num_cores=NC, num_subcores=NS
    )


def _agg_partials(h, src3d, dst3d, zeros):
    def body(h_hbm, src_hbm, dst_hbm, zero_hbm, out_hbm, src_v, dst_v, rows_v, acc,
             sg0, sg1, sg2, sg3, ss0, ss1, ss2, ss3):
        sg = [sg0, sg1, sg2, sg3]
        ss = [ss0, ss1, ss2, ss3]
        cid = lax.axis_index("c")
        sid = lax.axis_index("s")
        tid = cid * NS + sid
        for blk in range(NB):
            @pl.when(blk % NS == sid)
            def _():
                pltpu.sync_copy(zero_hbm.at[pl.ds(blk * RB, RB), :],
                                acc.at[pl.ds(blk * RB, RB), :])
        plsc.subcore_barrier()

        rows = [rows_v.at[k] for k in range(NBUF)]

        def drain(buf, s):
            pltpu.make_async_copy(h_hbm.at[pl.ds(0, CH), :], buf, s).wait()

        def run_round(off, cnt):
            pltpu.sync_copy(src_hbm.at[tid, pl.ds(off, cnt), :],
                            src_v.at[pl.ds(0, cnt), :])
            pltpu.sync_copy(dst_hbm.at[tid, pl.ds(off, cnt), :],
                            dst_v.at[pl.ds(0, cnt), :])
            for k in range(NBUF):
                pltpu.async_copy(h_hbm.at[src_v.at[k]], rows[k], sg[k])

            def group(g, carry):
                for k in range(NBUF):
                    c = NBUF * g + k
                    drain(rows[k], sg[k])
                    pltpu.async_copy(rows[k], acc.at[dst_v.at[c]], ss[k], add=True)
                for k in range(NBUF):
                    drain(rows[k], ss[k])
                    c_next = NBUF * g + NBUF + k

                    @pl.when(c_next < cnt)
                    def _():
                        pltpu.async_copy(h_hbm.at[src_v.at[c_next]], rows[k], sg[k])
                return carry

            lax.fori_loop(0, cnt // NBUF, group, 0)
            for r in range(cnt % NBUF):
                c = (cnt // NBUF) * NBUF + r
                drain(rows[r], sg[r])
                pltpu.async_copy(rows[r], acc.at[dst_v.at[c]], ss[r], add=True)
                drain(rows[r], ss[r])

        off = 0
        while off < TCH:
            cnt = min(IDXR, TCH - off)
            run_round(off, cnt)
            off += cnt
        plsc.subcore_barrier()
        for blk in range(NB):
            @pl.when(blk % NS == sid)
            def _():
                pltpu.sync_copy(acc.at[pl.ds(blk * RB, RB), :],
                                out_hbm.at[cid, pl.ds(blk * RB, RB), :])

    return pl.kernel(
        body,
        out_type=jax.ShapeDtypeStruct((NC, N, D), jnp.float32),
        mesh=_sc_mesh(),
        scratch_types=[
            pltpu.VMEM((IDXR, CH), jnp.int32),
            pltpu.VMEM((IDXR, CH), jnp.int32),
            pltpu.VMEM((NBUF, CH, D), jnp.float32),
            pltpu.VMEM_SHARED((N, D), jnp.float32),
        ] + [pltpu.SemaphoreType.DMA] * (2 * NBUF),
    )(h, src3d, dst3d, zeros)


def _deg_partials(dst3d, ones, zeros16):
    def body(dst_hbm, ones_hbm, zero_hbm, out_hbm, dst_v, ones_v, acc, sem):
        cid = lax.axis_index("c")
        sid = lax.axis_index("s")
        tid = cid * NS + sid
        for blk in range(NB):
            @pl.when(blk % NS == sid)
            def _():
                pltpu.sync_copy(zero_hbm.at[pl.ds(blk * RB, RB), :],
                                acc.at[pl.ds(blk * RB, RB), :])
        pltpu.sync_copy(ones_hbm, ones_v)
        pltpu.sync_copy(dst_hbm.at[tid], dst_v)
        plsc.subcore_barrier()

        def step(j, carry):
            pltpu.sync_copy(ones_v, acc.at[dst_v.at[j]], add=True)
            return carry

        lax.fori_loop(0, TCH, step, 0)
        plsc.subcore_barrier()
        for blk in range(NB):
            @pl.when(blk % NS == sid)
            def _():
                pltpu.sync_copy(acc.at[pl.ds(blk * RB, RB), :],
                                out_hbm.at[cid, pl.ds(blk * RB, RB), :])

    return pl.kernel(
        body,
        out_type=jax.ShapeDtypeStruct((NC, N, DEGW), jnp.float32),
        mesh=_sc_mesh(),
        scratch_types=[
            pltpu.VMEM((TCH, CH), jnp.int32),
            pltpu.VMEM((CH, DEGW), jnp.float32),
            pltpu.VMEM_SHARED((N, DEGW), jnp.float32),
            pltpu.SemaphoreType.DMA,
        ],
    )(dst3d, ones, zeros16)


def _dense_layer(h, accp, degp, Ws, Wn, b2d, relu):
    BLK = 400
    G = N // BLK

    def body(h_ref, a0, a1, d0, d1, ws, wn, b, out):
        deg = d0[0][:, 0:1] + d1[0][:, 0:1]
        inv = 1.0 / jnp.maximum(deg, 1.0)
        mean = (a0[0] + a1[0]) * inv
        y = jnp.dot(h_ref[...], ws[...], preferred_element_type=jnp.float32)
        y = y + jnp.dot(mean, wn[...], preferred_element_type=jnp.float32)
        y = y + b[...]
        if relu:
            y = jnp.maximum(y, 0.0)
        out[...] = y

    return pl.pallas_call(
        body,
        grid=(G,),
        in_specs=[
            pl.BlockSpec((BLK, D), lambda i: (i, 0)),
            pl.BlockSpec((1, BLK, D), lambda i: (0, i, 0)),
            pl.BlockSpec((1, BLK, D), lambda i: (1, i, 0)),
            pl.BlockSpec((1, BLK, DEGW), lambda i: (0, i, 0)),
            pl.BlockSpec((1, BLK, DEGW), lambda i: (1, i, 0)),
            pl.BlockSpec((D, D), lambda i: (0, 0)),
            pl.BlockSpec((D, D), lambda i: (0, 0)),
            pl.BlockSpec((1, D), lambda i: (0, 0)),
        ],
        out_specs=pl.BlockSpec((BLK, D), lambda i: (i, 0)),
        out_shape=jax.ShapeDtypeStruct((N, D), jnp.float32),
    )(h, accp, accp, degp, degp, Ws, Wn, b2d)


def kernel(x, edge_index, W_self_0, W_neigh_0, b_0, W_self_1, W_neigh_1, b_1,
           W_self_2, W_neigh_2, b_2):
    src3d = edge_index[0].reshape(NT, TCH, CH)
    dst3d = edge_index[1].reshape(NT, TCH, CH)
    zeros = jnp.zeros((N, D), jnp.float32)
    zeros16 = jnp.zeros((N, DEGW), jnp.float32)
    ones = jnp.ones((CH, DEGW), jnp.float32)
    degp = _deg_partials(dst3d, ones, zeros16)
    params = [
        (W_self_0, W_neigh_0, b_0),
        (W_self_1, W_neigh_1, b_1),
        (W_self_2, W_neigh_2, b_2),
    ]
    h = x
    for l, (Ws, Wn, b) in enumerate(params):
        accp = _agg_partials(h, src3d, dst3d, zeros)
        h = _dense_layer(h, accp, degp, Ws, Wn, b.reshape(1, D), relu=(l < 2))
    return h


# speedup vs baseline: 9.1198x; 1.0036x over previous
import jax
import jax.numpy as jnp
from jax import lax
from jax.experimental import pallas as pl
from jax.experimental.pallas import tpu as pltpu
from jax.experimental.pallas import tpu_sc as plsc

N = 10000
D = 128
E = 320000
NC = 2
NS = 16
CH = 80
NCH = E // CH
NT = NC * NS
TCH = NCH // NT
IDXR = 32
NBUF = 4
RB = 400
NB = N // RB
DEGW = 128


def _sc_mesh():
    return plsc.VectorSubcoreMesh(
        core_axis_name="c", subcore_axis_name="s", num_cores=NC, num_subcores=NS
    )


def _agg_partials(h, src3d, dst3d, zeros):
    def body(h_hbm, src_hbm, dst_hbm, zero_hbm, out_hbm, src_v, dst_v, rows_v, acc,
             sg0, sg1, sg2, sg3, ss0, ss1, ss2, ss3):
        sg = [sg0, sg1, sg2, sg3]
        ss = [ss0, ss1, ss2, ss3]
        cid = lax.axis_index("c")
        sid = lax.axis_index("s")
        tid = cid * NS + sid
        for blk in range(NB):
            @pl.when(blk % NS == sid)
            def _():
                pltpu.sync_copy(zero_hbm.at[pl.ds(blk * RB, RB), :],
                                acc.at[pl.ds(blk * RB, RB), :])
        plsc.subcore_barrier()

        rows = [rows_v.at[k] for k in range(NBUF)]

        def drain(buf, s):
            pltpu.make_async_copy(h_hbm.at[pl.ds(0, CH), :], buf, s).wait()

        def run_round(off, cnt):
            pltpu.sync_copy(src_hbm.at[tid, pl.ds(off, cnt), :],
                            src_v.at[pl.ds(0, cnt), :])
            pltpu.sync_copy(dst_hbm.at[tid, pl.ds(off, cnt), :],
                            dst_v.at[pl.ds(0, cnt), :])
            for k in range(NBUF):
                pltpu.async_copy(h_hbm.at[src_v.at[k]], rows[k], sg[k])

            def group(g, carry):
                for k in range(NBUF):
                    c = NBUF * g + k
                    drain(rows[k], sg[k])
                    pltpu.async_copy(rows[k], acc.at[dst_v.at[c]], ss[k], add=True)
                for k in range(NBUF):
                    drain(rows[k], ss[k])
                    c_next = NBUF * g + NBUF + k

                    @pl.when(c_next < cnt)
                    def _():
                        pltpu.async_copy(h_hbm.at[src_v.at[c_next]], rows[k], sg[k])
                return carry

            lax.fori_loop(0, cnt // NBUF, group, 0)
            for r in range(cnt % NBUF):
                c = (cnt // NBUF) * NBUF + r
                drain(rows[r], sg[r])
                pltpu.async_copy(rows[r], acc.at[dst_v.at[c]], ss[r], add=True)
                drain(rows[r], ss[r])

        off = 0
        while off < TCH:
            cnt = min(IDXR, TCH - off)
            run_round(off, cnt)
            off += cnt
        plsc.subcore_barrier()
        for blk in range(NB):
            @pl.when(blk % NS == sid)
            def _():
                pltpu.sync_copy(acc.at[pl.ds(blk * RB, RB), :],
                                out_hbm.at[cid, pl.ds(blk * RB, RB), :])

    return pl.kernel(
        body,
        out_type=jax.ShapeDtypeStruct((NC, N, D), jnp.float32),
        mesh=_sc_mesh(),
        scratch_types=[
            pltpu.VMEM((IDXR, CH), jnp.int32),
            pltpu.VMEM((IDXR, CH), jnp.int32),
            pltpu.VMEM((NBUF, CH, D), jnp.float32),
            pltpu.VMEM_SHARED((N, D), jnp.float32),
        ] + [pltpu.SemaphoreType.DMA] * (2 * NBUF),
    )(h, src3d, dst3d, zeros)


def _deg_partials(dst3d, ones, zeros16):
    def body(dst_hbm, ones_hbm, zero_hbm, out_hbm, dst_v, ones_v, acc,
             ss0, ss1, ss2, ss3):
        ss = [ss0, ss1, ss2, ss3]
        cid = lax.axis_index("c")
        sid = lax.axis_index("s")
        tid = cid * NS + sid
        for blk in range(NB):
            @pl.when(blk % NS == sid)
            def _():
                pltpu.sync_copy(zero_hbm.at[pl.ds(blk * RB, RB), :],
                                acc.at[pl.ds(blk * RB, RB), :])
        pltpu.sync_copy(ones_hbm, ones_v)
        pltpu.sync_copy(dst_hbm.at[tid], dst_v)
        plsc.subcore_barrier()

        def drain(s):
            pltpu.make_async_copy(ones_hbm, ones_v, s).wait()

        for k in range(NBUF):
            pltpu.async_copy(ones_v, acc.at[dst_v.at[k]], ss[k], add=True)

        def group(g, carry):
            for k in range(NBUF):
                drain(ss[k])
                c_next = NBUF * g + NBUF + k

                @pl.when(c_next < TCH)
                def _():
                    pltpu.async_copy(ones_v, acc.at[dst_v.at[c_next]], ss[k],
                                     add=True)
            return carry

        F = (TCH - 1) // NBUF
        lax.fori_loop(0, F, group, 0)
        for k in range(NBUF):
            issued = 1 + sum(1 for g in range(F) if NBUF * g + NBUF + k < TCH)
            for _ in range(issued - F):
                drain(ss[k])
        plsc.subcore_barrier()
        for blk in range(NB):
            @pl.when(blk % NS == sid)
            def _():
                pltpu.sync_copy(acc.at[pl.ds(blk * RB, RB), :],
                                out_hbm.at[cid, pl.ds(blk * RB, RB), :])

    return pl.kernel(
        body,
        out_type=jax.ShapeDtypeStruct((NC, N, DEGW), jnp.float32),
        mesh=_sc_mesh(),
        scratch_types=[
            pltpu.VMEM((TCH, CH), jnp.int32),
            pltpu.VMEM((CH, DEGW), jnp.float32),
            pltpu.VMEM_SHARED((N, DEGW), jnp.float32),
        ] + [pltpu.SemaphoreType.DMA] * NBUF,
    )(dst3d, ones, zeros16)


def _dense_layer(h, accp, degp, Ws, Wn, b2d, relu):
    BLK = 400
    G = N // BLK

    def body(h_ref, a0, a1, d0, d1, ws, wn, b, out):
        deg = d0[0][:, 0:1] + d1[0][:, 0:1]
        inv = 1.0 / jnp.maximum(deg, 1.0)
        mean = (a0[0] + a1[0]) * inv
        y = jnp.dot(h_ref[...], ws[...], preferred_element_type=jnp.float32)
        y = y + jnp.dot(mean, wn[...], preferred_element_type=jnp.float32)
        y = y + b[...]
        if relu:
            y = jnp.maximum(y, 0.0)
        out[...] = y

    return pl.pallas_call(
        body,
        grid=(G,),
        in_specs=[
            pl.BlockSpec((BLK, D), lambda i: (i, 0)),
            pl.BlockSpec((1, BLK, D), lambda i: (0, i, 0)),
            pl.BlockSpec((1, BLK, D), lambda i: (1, i, 0)),
            pl.BlockSpec((1, BLK, DEGW), lambda i: (0, i, 0)),
            pl.BlockSpec((1, BLK, DEGW), lambda i: (1, i, 0)),
            pl.BlockSpec((D, D), lambda i: (0, 0)),
            pl.BlockSpec((D, D), lambda i: (0, 0)),
            pl.BlockSpec((1, D), lambda i: (0, 0)),
        ],
        out_specs=pl.BlockSpec((BLK, D), lambda i: (i, 0)),
        out_shape=jax.ShapeDtypeStruct((N, D), jnp.float32),
    )(h, accp, accp, degp, degp, Ws, Wn, b2d)


def kernel(x, edge_index, W_self_0, W_neigh_0, b_0, W_self_1, W_neigh_1, b_1,
           W_self_2, W_neigh_2, b_2):
    src3d = edge_index[0].reshape(NT, TCH, CH)
    dst3d = edge_index[1].reshape(NT, TCH, CH)
    zeros = jnp.zeros((N, D), jnp.float32)
    zeros16 = jnp.zeros((N, DEGW), jnp.float32)
    ones = jnp.ones((CH, DEGW), jnp.float32)
    degp = _deg_partials(dst3d, ones, zeros16)
    params = [
        (W_self_0, W_neigh_0, b_0),
        (W_self_1, W_neigh_1, b_1),
        (W_self_2, W_neigh_2, b_2),
    ]
    h = x
    for l, (Ws, Wn, b) in enumerate(params):
        accp = _agg_partials(h, src3d, dst3d, zeros)
        h = _dense_layer(h, accp, degp, Ws, Wn, b.reshape(1, D), relu=(l < 2))
    return h


# speedup vs baseline: 9.1518x; 1.0035x over previous
import jax
import jax.numpy as jnp
from jax import lax
from jax.experimental import pallas as pl
from jax.experimental.pallas import tpu as pltpu
from jax.experimental.pallas import tpu_sc as plsc

N = 10000
D = 128
E = 320000
NC = 2
NS = 16
CH = 80
NCH = E // CH
NT = NC * NS
TCH = NCH // NT
IDXR = 16
NBUF = 4
RB = 400
NB = N // RB
DEGW = 128


def _sc_mesh():
    return plsc.VectorSubcoreMesh(
        core_axis_name="c", subcore_axis_name="s", num_cores=NC, num_subcores=NS
    )


def _agg_partials(h, src3d, dst3d, zeros):
    def body(h_hbm, src_hbm, dst_hbm, zero_hbm, out_hbm, src_v, dst_v, rows_v, acc,
             sg0, sg1, sg2, sg3, ss0, ss1, ss2, ss3, si0, si1):
        sg = [sg0, sg1, sg2, sg3]
        ss = [ss0, ss1, ss2, ss3]
        si = [si0, si1]
        cid = lax.axis_index("c")
        sid = lax.axis_index("s")
        tid = cid * NS + sid
        for blk in range(NB):
            @pl.when(blk % NS == sid)
            def _():
                pltpu.sync_copy(zero_hbm.at[pl.ds(blk * RB, RB), :],
                                acc.at[pl.ds(blk * RB, RB), :])
        plsc.subcore_barrier()

        rows = [rows_v.at[k] for k in range(NBUF)]

        def drain(buf, s):
            pltpu.make_async_copy(h_hbm.at[pl.ds(0, CH), :], buf, s).wait()

        rounds = []
        off = 0
        while off < TCH:
            rounds.append((off, min(IDXR, TCH - off)))
            off += rounds[-1][1]

        def load_idx(r, start):
            off, cnt = rounds[r]
            slot = r % 2
            dsc_s = pltpu.make_async_copy(src_hbm.at[tid, pl.ds(off, cnt), :],
                                          src_v.at[slot, pl.ds(0, cnt), :],
                                          si[slot])
            dsc_d = pltpu.make_async_copy(dst_hbm.at[tid, pl.ds(off, cnt), :],
                                          dst_v.at[slot, pl.ds(0, cnt), :],
                                          si[slot])
            if start:
                dsc_s.start()
                dsc_d.start()
            else:
                dsc_s.wait()
                dsc_d.wait()

        def run_round(r):
            _, cnt = rounds[r]
            slot = r % 2
            sv = src_v.at[slot]
            dv = dst_v.at[slot]
            load_idx(r, start=False)
            if r + 1 < len(rounds):
                load_idx(r + 1, start=True)
            for k in range(NBUF):
                pltpu.async_copy(h_hbm.at[sv.at[k]], rows[k], sg[k])

            def group(g, carry):
                for k in range(NBUF):
                    c = NBUF * g + k
                    drain(rows[k], sg[k])
                    pltpu.async_copy(rows[k], acc.at[dv.at[c]], ss[k], add=True)
                for k in range(NBUF):
                    drain(rows[k], ss[k])
                    c_next = NBUF * g + NBUF + k

                    @pl.when(c_next < cnt)
                    def _():
                        pltpu.async_copy(h_hbm.at[sv.at[c_next]], rows[k], sg[k])
                return carry

            lax.fori_loop(0, cnt // NBUF, group, 0)
            for q in range(cnt % NBUF):
                c = (cnt // NBUF) * NBUF + q
                drain(rows[q], sg[q])
                pltpu.async_copy(rows[q], acc.at[dv.at[c]], ss[q], add=True)
                drain(rows[q], ss[q])

        load_idx(0, start=True)
        for r in range(len(rounds)):
            run_round(r)
        plsc.subcore_barrier()
        for blk in range(NB):
            @pl.when(blk % NS == sid)
            def _():
                pltpu.sync_copy(acc.at[pl.ds(blk * RB, RB), :],
                                out_hbm.at[cid, pl.ds(blk * RB, RB), :])

    return pl.kernel(
        body,
        out_type=jax.ShapeDtypeStruct((NC, N, D), jnp.float32),
        mesh=_sc_mesh(),
        scratch_types=[
            pltpu.VMEM((2, IDXR, CH), jnp.int32),
            pltpu.VMEM((2, IDXR, CH), jnp.int32),
            pltpu.VMEM((NBUF, CH, D), jnp.float32),
            pltpu.VMEM_SHARED((N, D), jnp.float32),
        ] + [pltpu.SemaphoreType.DMA] * (2 * NBUF + 2),
    )(h, src3d, dst3d, zeros)


def _deg_partials(dst3d, ones, zeros16):
    def body(dst_hbm, ones_hbm, zero_hbm, out_hbm, dst_v, ones_v, acc,
             ss0, ss1, ss2, ss3):
        ss = [ss0, ss1, ss2, ss3]
        cid = lax.axis_index("c")
        sid = lax.axis_index("s")
        tid = cid * NS + sid
        for blk in range(NB):
            @pl.when(blk % NS == sid)
            def _():
                pltpu.sync_copy(zero_hbm.at[pl.ds(blk * RB, RB), :],
                                acc.at[pl.ds(blk * RB, RB), :])
        pltpu.sync_copy(ones_hbm, ones_v)
        pltpu.sync_copy(dst_hbm.at[tid], dst_v)
        plsc.subcore_barrier()

        def drain(s):
            pltpu.make_async_copy(ones_hbm, ones_v, s).wait()

        for k in range(NBUF):
            pltpu.async_copy(ones_v, acc.at[dst_v.at[k]], ss[k], add=True)

        def group(g, carry):
            for k in range(NBUF):
                drain(ss[k])
                c_next = NBUF * g + NBUF + k

                @pl.when(c_next < TCH)
                def _():
                    pltpu.async_copy(ones_v, acc.at[dst_v.at[c_next]], ss[k],
                                     add=True)
            return carry

        F = (TCH - 1) // NBUF
        lax.fori_loop(0, F, group, 0)
        for k in range(NBUF):
            issued = 1 + sum(1 for g in range(F) if NBUF * g + NBUF + k < TCH)
            for _ in range(issued - F):
                drain(ss[k])
        plsc.subcore_barrier()
        for blk in range(NB):
            @pl.when(blk % NS == sid)
            def _():
                pltpu.sync_copy(acc.at[pl.ds(blk * RB, RB), :],
                                out_hbm.at[cid, pl.ds(blk * RB, RB), :])

    return pl.kernel(
        body,
        out_type=jax.ShapeDtypeStruct((NC, N, DEGW), jnp.float32),
        mesh=_sc_mesh(),
        scratch_types=[
            pltpu.VMEM((TCH, CH), jnp.int32),
            pltpu.VMEM((CH, DEGW), jnp.float32),
            pltpu.VMEM_SHARED((N, DEGW), jnp.float32),
        ] + [pltpu.SemaphoreType.DMA] * NBUF,
    )(dst3d, ones, zeros16)


def _dense_layer(h, accp, degp, Ws, Wn, b2d, relu):
    BLK = 400
    G = N // BLK

    def body(h_ref, a0, a1, d0, d1, ws, wn, b, out):
        deg = d0[0][:, 0:1] + d1[0][:, 0:1]
        inv = 1.0 / jnp.maximum(deg, 1.0)
        mean = (a0[0] + a1[0]) * inv
        y = jnp.dot(h_ref[...], ws[...], preferred_element_type=jnp.float32)
        y = y + jnp.dot(mean, wn[...], preferred_element_type=jnp.float32)
        y = y + b[...]
        if relu:
            y = jnp.maximum(y, 0.0)
        out[...] = y

    return pl.pallas_call(
        body,
        grid=(G,),
        in_specs=[
            pl.BlockSpec((BLK, D), lambda i: (i, 0)),
            pl.BlockSpec((1, BLK, D), lambda i: (0, i, 0)),
            pl.BlockSpec((1, BLK, D), lambda i: (1, i, 0)),
            pl.BlockSpec((1, BLK, DEGW), lambda i: (0, i, 0)),
            pl.BlockSpec((1, BLK, DEGW), lambda i: (1, i, 0)),
            pl.BlockSpec((D, D), lambda i: (0, 0)),
            pl.BlockSpec((D, D), lambda i: (0, 0)),
            pl.BlockSpec((1, D), lambda i: (0, 0)),
        ],
        out_specs=pl.BlockSpec((BLK, D), lambda i: (i, 0)),
        out_shape=jax.ShapeDtypeStruct((N, D), jnp.float32),
    )(h, accp, accp, degp, degp, Ws, Wn, b2d)


def kernel(x, edge_index, W_self_0, W_neigh_0, b_0, W_self_1, W_neigh_1, b_1,
           W_self_2, W_neigh_2, b_2):
    src3d = edge_index[0].reshape(NT, TCH, CH)
    dst3d = edge_index[1].reshape(NT, TCH, CH)
    zeros = jnp.zeros((N, D), jnp.float32)
    zeros16 = jnp.zeros((N, DEGW), jnp.float32)
    ones = jnp.ones((CH, DEGW), jnp.float32)
    degp = _deg_partials(dst3d, ones, zeros16)
    params = [
        (W_self_0, W_neigh_0, b_0),
        (W_self_1, W_neigh_1, b_1),
        (W_self_2, W_neigh_2, b_2),
    ]
    h = x
    for l, (Ws, Wn, b) in enumerate(params):
        accp = _agg_partials(h, src3d, dst3d, zeros)
        h = _dense_layer(h, accp, degp, Ws, Wn, b.reshape(1, D), relu=(l < 2))
    return h
